# baseline (speedup 1.0000x reference)
import functools

import jax
import jax.numpy as jnp
from jax import lax
from jax.experimental import pallas as pl
from jax.experimental.pallas import tpu as pltpu
from jax.experimental.pallas import tpu_sc as plsc

NC = 2
NS = 16
LANES = 16
SR = 4


def kernel(words, word_unk_label, W, W_unk):
    B, N = words.shape
    V, D = W.shape
    NW = NC * NS
    BPT = B // NW
    LG = BPT // LANES
    assert B == NW * BPT and BPT == 128
    assert D == 2 * LANES and V % SR == 0
    assert N % 2 == 0

    words_t = words.astype(jnp.int32).T
    unk_t = word_unk_label.astype(jnp.int32).T
    w128 = W.reshape(V // SR, D * SR)

    @functools.partial(
        pl.kernel,
        mesh=plsc.VectorSubcoreMesh(core_axis_name="c", subcore_axis_name="s"),
        out_type=jax.ShapeDtypeStruct((N, D, B), jnp.float32),
        compiler_params=pltpu.CompilerParams(needs_layout_passes=False),
        scratch_types=[
            pltpu.VMEM((N, BPT), jnp.int32),
            pltpu.VMEM((N, BPT), jnp.int32),
            pltpu.VMEM((4, BPT), jnp.int32),
            pltpu.VMEM((4, BPT, D * SR), jnp.float32),
            pltpu.VMEM((4, D, BPT), jnp.float32),
            pltpu.VMEM((1, D), jnp.float32),
            pltpu.VMEM((LANES, D), jnp.float32),
            pltpu.VMEM((D, LANES), jnp.float32),
            pltpu.SemaphoreType.DMA,
            pltpu.SemaphoreType.DMA,
            pltpu.SemaphoreType.DMA,
            pltpu.SemaphoreType.DMA,
            pltpu.SemaphoreType.DMA,
            pltpu.SemaphoreType.DMA,
            pltpu.SemaphoreType.DMA,
            pltpu.SemaphoreType.DMA,
        ],
    )
    def _emb(words_hbm, unk_hbm, w_hbm, wunk_hbm, out_hbm,
             idx_v, unk_v, sidx_v, rows_v, asm_v, wrow_v, wrep_v, wb_v,
             gsem0, gsem1, gsem2, gsem3, wsem0, wsem1, wsem2, wsem3):
        gsems = [gsem0, gsem1, gsem2, gsem3]
        wsems = [wsem0, wsem1, wsem2, wsem3]
        wid = lax.axis_index("s") * NC + lax.axis_index("c")
        col0 = pl.multiple_of(wid * BPT, BPT)

        pltpu.sync_copy(words_hbm.at[:, pl.ds(col0, BPT)], idx_v)
        pltpu.sync_copy(unk_hbm.at[:, pl.ds(col0, BPT)], unk_v)
        pltpu.sync_copy(wunk_hbm, wrow_v)

        iota = lax.iota(jnp.int32, LANES)
        lo = wrow_v[0, pl.ds(0, LANES)]
        hi = wrow_v[0, pl.ds(LANES, LANES)]
        for r in range(LANES):
            wrep_v[r, pl.ds(0, LANES)] = lo
            wrep_v[r, pl.ds(LANES, LANES)] = hi
        for d in range(D):
            wb_v[d, :] = plsc.load_gather(wrep_v, [iota, iota * 0 + d])

        lanes = [l * LANES + iota for l in range(LG)]

        def gather_start(p, n):
            for l in range(LG):
                iv = idx_v[n, pl.ds(l * LANES, LANES)]
                sidx_v[p, pl.ds(l * LANES, LANES)] = lax.shift_right_logical(iv, 2)
            pltpu.async_copy(w_hbm.at[sidx_v.at[p]], rows_v.at[p], gsems[p])

        def gather_wait(p):
            pltpu.make_async_copy(
                w_hbm.at[sidx_v.at[p]], rows_v.at[p], gsems[p]
            ).wait()

        def assemble(p, n):
            offs = []
            masks = []
            for l in range(LG):
                iv = idx_v[n, pl.ds(l * LANES, LANES)]
                offs.append(lax.shift_left(iv & (SR - 1), 5))
                masks.append(unk_v[n, pl.ds(l * LANES, LANES)] != 0)
            for d in range(D):
                wbd = wb_v[d, :]
                for l in range(LG):
                    v = plsc.load_gather(rows_v.at[p], [lanes[l], offs[l] + d])
                    asm_v[p, d, pl.ds(l * LANES, LANES)] = jnp.where(masks[l], wbd, v)

        def write_start(p, n):
            pltpu.async_copy(
                asm_v.at[p], out_hbm.at[n, :, pl.ds(col0, BPT)], wsems[p]
            )

        def write_wait(p, n):
            pltpu.make_async_copy(
                asm_v.at[p], out_hbm.at[n, :, pl.ds(col0, BPT)], wsems[p]
            ).wait()

        NQ = (N // 4) * 4

        for b in range(4):
            gather_start(b, b)

        def qbody(q, carry):
            for b in range(4):
                n = q * 4 + b
                gather_wait(b)

                @pl.when(q > 0)
                def _():
                    write_wait(b, n)

                assemble(b, n)
                write_start(b, n)

                @pl.when(n + 4 < NQ)
                def _():
                    gather_start(b, n + 4)

            return carry

        lax.fori_loop(0, NQ // 4, qbody, 0)

        for b in range(N - NQ):
            gather_start(b, NQ + b)
        for b in range(N - NQ):
            gather_wait(b)
            write_wait(b, 0)
            assemble(b, NQ + b)
            write_start(b, NQ + b)
        for b in range(N - NQ, 4):
            write_wait(b, 0)
        for b in range(N - NQ):
            write_wait(b, 0)

    out = _emb(words_t, unk_t, w128, W_unk)
    return jnp.transpose(out, (2, 0, 1))


# speedup vs baseline: 1.0265x; 1.0265x over previous
import functools

import jax
import jax.numpy as jnp
from jax import lax
from jax.experimental import pallas as pl
from jax.experimental.pallas import tpu as pltpu
from jax.experimental.pallas import tpu_sc as plsc

NC = 2
NS = 16
LANES = 16
SR = 4


def kernel(words, word_unk_label, W, W_unk):
    B, N = words.shape
    V, D = W.shape
    NW = NC * NS
    BPT = B // NW
    LG = BPT // LANES
    assert B == NW * BPT and BPT == 128
    assert D == 2 * LANES and V % SR == 0
    assert N % 2 == 0

    words_t = words.astype(jnp.int32).T
    unk_t = word_unk_label.astype(jnp.int32).T
    w128 = W.reshape(V // SR, D * SR)

    @functools.partial(
        pl.kernel,
        mesh=plsc.VectorSubcoreMesh(core_axis_name="c", subcore_axis_name="s"),
        out_type=jax.ShapeDtypeStruct((N, D, B), jnp.float32),
        compiler_params=pltpu.CompilerParams(needs_layout_passes=False),
        scratch_types=[
            pltpu.VMEM((N, BPT), jnp.int32),
            pltpu.VMEM((N, BPT), jnp.int32),
            pltpu.VMEM((2, BPT), jnp.int32),
            pltpu.VMEM((2, BPT, D * SR), jnp.float32),
            pltpu.VMEM((2, D, BPT), jnp.float32),
            pltpu.VMEM((1, D), jnp.float32),
            pltpu.VMEM((LANES, D), jnp.float32),
            pltpu.VMEM((D, LANES), jnp.float32),
            pltpu.SemaphoreType.DMA,
            pltpu.SemaphoreType.DMA,
            pltpu.SemaphoreType.DMA,
            pltpu.SemaphoreType.DMA,
        ],
    )
    def _emb(words_hbm, unk_hbm, w_hbm, wunk_hbm, out_hbm,
             idx_v, unk_v, sidx_v, rows_v, asm_v, wrow_v, wrep_v, wb_v,
             gsem0, gsem1, wsem0, wsem1):
        gsems = [gsem0, gsem1]
        wsems = [wsem0, wsem1]
        wid = lax.axis_index("s") * NC + lax.axis_index("c")
        col0 = pl.multiple_of(wid * BPT, BPT)

        pltpu.sync_copy(words_hbm.at[:, pl.ds(col0, BPT)], idx_v)
        pltpu.sync_copy(unk_hbm.at[:, pl.ds(col0, BPT)], unk_v)
        pltpu.sync_copy(wunk_hbm, wrow_v)

        iota = lax.iota(jnp.int32, LANES)
        lo = wrow_v[0, pl.ds(0, LANES)]
        hi = wrow_v[0, pl.ds(LANES, LANES)]
        for r in range(LANES):
            wrep_v[r, pl.ds(0, LANES)] = lo
            wrep_v[r, pl.ds(LANES, LANES)] = hi
        for d in range(D):
            wb_v[d, :] = plsc.load_gather(wrep_v, [iota, iota * 0 + d])

        lanes = [l * LANES + iota for l in range(LG)]

        def gather_start(p, n):
            for l in range(LG):
                iv = idx_v[n, pl.ds(l * LANES, LANES)]
                sidx_v[p, pl.ds(l * LANES, LANES)] = lax.shift_right_logical(iv, 2)
            pltpu.async_copy(w_hbm.at[sidx_v.at[p]], rows_v.at[p], gsems[p])

        def gather_wait(p):
            pltpu.make_async_copy(
                w_hbm.at[sidx_v.at[p]], rows_v.at[p], gsems[p]
            ).wait()

        def assemble(p, n):
            offs = []
            masks = []
            for l in range(LG):
                iv = idx_v[n, pl.ds(l * LANES, LANES)]
                offs.append(lax.shift_left(iv & (SR - 1), 5))
                masks.append(unk_v[n, pl.ds(l * LANES, LANES)] != 0)
            for d in range(D):
                wbd = wb_v[d, :]
                for l in range(LG):
                    v = plsc.load_gather(rows_v.at[p], [lanes[l], offs[l] + d])
                    asm_v[p, d, pl.ds(l * LANES, LANES)] = jnp.where(masks[l], wbd, v)

        def write_start(p, n):
            pltpu.async_copy(
                asm_v.at[p], out_hbm.at[n, :, pl.ds(col0, BPT)], wsems[p]
            )

        def write_wait(p, n):
            pltpu.make_async_copy(
                asm_v.at[p], out_hbm.at[n, :, pl.ds(col0, BPT)], wsems[p]
            ).wait()

        gather_start(0, 0)

        def gbody(g, carry):
            n0 = g * 2
            n1 = n0 + 1
            gather_start(1, n1)
            gather_wait(0)

            @pl.when(g > 0)
            def _():
                write_wait(0, n0)

            assemble(0, n0)
            write_start(0, n0)
            @pl.when(g + 1 < N // 2)
            def _():
                gather_start(0, n0 + 2)

            gather_wait(1)

            @pl.when(g > 0)
            def _():
                write_wait(1, n1)

            assemble(1, n1)
            write_start(1, n1)
            return carry

        lax.fori_loop(0, N // 2, gbody, 0)
        write_wait(0, 0)
        write_wait(1, 0)

    out = _emb(words_t, unk_t, w128, W_unk)
    return jnp.transpose(out, (2, 0, 1))
